# baseline (speedup 1.0000x reference)
import functools

import jax
import jax.numpy as jnp
from jax import lax
from jax.experimental import pallas as pl
from jax.experimental.pallas import tpu as pltpu
from jax.experimental.pallas import tpu_sc as plsc

L = 16
NBUF = 2
TB = 4


def _build_sc_call(n_tokens, vocab, d_model, nc, ns):
    nw = nc * ns
    per_w = n_tokens // nw
    chunk = 400
    while per_w % chunk:
        chunk //= 2
    n_chunks = per_w // chunk
    n_groups = chunk // L
    n_slices = d_model // L

    mesh = plsc.VectorSubcoreMesh(core_axis_name="c", subcore_axis_name="s",
                                  num_cores=nc, num_subcores=ns)

    @functools.partial(
        pl.kernel,
        out_type=jax.ShapeDtypeStruct((n_tokens * d_model,), jnp.float32),
        mesh=mesh,
        scratch_types=[
            pltpu.VMEM((vocab * d_model,), jnp.float32),
            [pltpu.VMEM((chunk,), jnp.int32)
             for _ in range(NBUF)],
            [pltpu.VMEM((chunk,), jnp.float32)
             for _ in range(NBUF)],
            [pltpu.VMEM((chunk * d_model,), jnp.float32)
             for _ in range(NBUF)],
            [pltpu.SemaphoreType.DMA for _ in range(NBUF)],
            [pltpu.SemaphoreType.DMA for _ in range(NBUF)],
        ],
        compiler_params=pltpu.CompilerParams(needs_layout_passes=False),
    )
    def emb_kernel(x_hbm, mask_hbm, table_hbm, out_hbm,
                   table_v, idx_bufs, mask_bufs, out_bufs, out_sems, in_sems):
        wid = lax.axis_index("s") * nc + lax.axis_index("c")
        base = wid * per_w
        pltpu.sync_copy(table_hbm, table_v)
        lane = lax.iota(jnp.int32, L)
        soff = [s * L + lane for s in range(n_slices)]
        dnums = lax.GatherDimensionNumbers(
            offset_dims=(), collapsed_slice_dims=(0,), start_index_map=(0,))

        def lane_bcast(vec, j):
            jvec = jnp.full((L, 1), j, jnp.int32)
            return lax.gather(vec, jvec, dnums, (1,),
                              mode=lax.GatherScatterMode.PROMISE_IN_BOUNDS)

        def start_loads(ci, b):
            start = base + ci * chunk
            pltpu.async_copy(x_hbm.at[pl.ds(start, chunk)],
                             idx_bufs[b], in_sems[b])
            pltpu.async_copy(mask_hbm.at[pl.ds(start, chunk)],
                             mask_bufs[b], in_sems[b])

        def wait_loads(ci, b):
            start = base + ci * chunk
            pltpu.make_async_copy(x_hbm.at[pl.ds(start, chunk)],
                                  idx_bufs[b], in_sems[b]).wait()
            pltpu.make_async_copy(mask_hbm.at[pl.ds(start, chunk)],
                                  mask_bufs[b], in_sems[b]).wait()

        def do_chunk(ci, b, buf, sem):
            start = base + ci * chunk
            idx_v = idx_bufs[b]
            mask_v = mask_bufs[b]

            @plsc.parallel_loop(0, n_groups, 1, unroll=2)
            def group_body(g):
                tok0 = g * L
                rows16 = idx_v[pl.ds(tok0, L)] * d_model
                msk16 = mask_v[pl.ds(tok0, L)]
                for j0 in range(0, L, TB):
                    cols = []
                    for j in range(j0, j0 + TB):
                        row_s = lane_bcast(rows16, j)
                        for s in range(n_slices):
                            cols.append(
                                plsc.load_gather(table_v, [row_s + soff[s]]))
                    k = 0
                    for j in range(j0, j0 + TB):
                        msk_s = lane_bcast(msk16, j)
                        tbase = (tok0 + j) * d_model
                        for s in range(n_slices):
                            buf[pl.ds(tbase + s * L, L)] = cols[k] * msk_s
                            k += 1
            pltpu.async_copy(
                buf, out_hbm.at[pl.ds(start * d_model, chunk * d_model)], sem)

        for b in range(NBUF):
            start_loads(b, b)

        def outer(i, carry):
            for b in range(NBUF):
                ci = i * NBUF + b
                wait_loads(ci, b)

                @pl.when(ci >= NBUF)
                def _():
                    prev = base + (ci - NBUF) * chunk
                    pltpu.make_async_copy(
                        out_bufs[b],
                        out_hbm.at[pl.ds(prev * d_model, chunk * d_model)],
                        out_sems[b]).wait()

                do_chunk(ci, b, out_bufs[b], out_sems[b])

                @pl.when(ci + NBUF < n_chunks)
                def _():
                    start_loads(ci + NBUF, b)
            return carry

        lax.fori_loop(0, n_chunks // NBUF, outer, 0)
        for b in range(NBUF):
            last = base + (n_chunks - NBUF + b) * chunk
            pltpu.make_async_copy(
                out_bufs[b],
                out_hbm.at[pl.ds(last * d_model, chunk * d_model)],
                out_sems[b]).wait()

    return emb_kernel


def kernel(x, mask, emb_table):
    b, h = x.shape
    vocab, d_model = emb_table.shape
    n = b * h
    x_flat = x.T.reshape(n).astype(jnp.int32)
    mask_flat = mask.T.reshape(n).astype(jnp.float32)
    table_flat = emb_table.reshape(vocab * d_model).astype(jnp.float32)
    info = plsc.get_sparse_core_info()
    call = _build_sc_call(n, vocab, d_model, info.num_cores, info.num_subcores)
    out = call(x_flat, mask_flat, table_flat)
    return out.reshape(h, b, d_model).transpose(1, 0, 2)


# speedup vs baseline: 1.0528x; 1.0528x over previous
import functools

import jax
import jax.numpy as jnp
from jax import lax
from jax.experimental import pallas as pl
from jax.experimental.pallas import tpu as pltpu
from jax.experimental.pallas import tpu_sc as plsc

L = 16
NBUF = 2
TB = 2


def _build_sc_call(n_tokens, vocab, d_model, nc, ns):
    nw = nc * ns
    per_w = n_tokens // nw
    chunk = 400
    while per_w % chunk:
        chunk //= 2
    n_chunks = per_w // chunk
    n_groups = chunk // L
    n_slices = d_model // L

    mesh = plsc.VectorSubcoreMesh(core_axis_name="c", subcore_axis_name="s",
                                  num_cores=nc, num_subcores=ns)

    @functools.partial(
        pl.kernel,
        out_type=jax.ShapeDtypeStruct((n_tokens * d_model,), jnp.float32),
        mesh=mesh,
        scratch_types=[
            pltpu.VMEM((vocab * d_model,), jnp.float32),
            [pltpu.VMEM((chunk,), jnp.int32)
             for _ in range(NBUF)],
            [pltpu.VMEM((chunk,), jnp.float32)
             for _ in range(NBUF)],
            [pltpu.VMEM((chunk * d_model,), jnp.float32)
             for _ in range(NBUF)],
            [pltpu.SemaphoreType.DMA for _ in range(NBUF)],
            [pltpu.SemaphoreType.DMA for _ in range(NBUF)],
        ],
        compiler_params=pltpu.CompilerParams(needs_layout_passes=False),
    )
    def emb_kernel(x_hbm, mask_hbm, table_hbm, out_hbm,
                   table_v, idx_bufs, mask_bufs, out_bufs, out_sems, in_sems):
        wid = lax.axis_index("s") * nc + lax.axis_index("c")
        base = wid * per_w
        pltpu.sync_copy(table_hbm, table_v)
        lane = lax.iota(jnp.int32, L)
        soff = [s * L + lane for s in range(n_slices)]
        dnums = lax.GatherDimensionNumbers(
            offset_dims=(), collapsed_slice_dims=(0,), start_index_map=(0,))

        def lane_bcast(vec, j):
            jvec = jnp.full((L, 1), j, jnp.int32)
            return lax.gather(vec, jvec, dnums, (1,),
                              mode=lax.GatherScatterMode.PROMISE_IN_BOUNDS)

        def start_loads(ci, b):
            start = base + ci * chunk
            pltpu.async_copy(x_hbm.at[pl.ds(start, chunk)],
                             idx_bufs[b], in_sems[b])
            pltpu.async_copy(mask_hbm.at[pl.ds(start, chunk)],
                             mask_bufs[b], in_sems[b])

        def wait_loads(ci, b):
            start = base + ci * chunk
            pltpu.make_async_copy(x_hbm.at[pl.ds(start, chunk)],
                                  idx_bufs[b], in_sems[b]).wait()
            pltpu.make_async_copy(mask_hbm.at[pl.ds(start, chunk)],
                                  mask_bufs[b], in_sems[b]).wait()

        def do_chunk(ci, b, buf, sem):
            start = base + ci * chunk
            idx_v = idx_bufs[b]
            mask_v = mask_bufs[b]

            @plsc.parallel_loop(0, n_groups, 1)
            def group_body(g):
                tok0 = g * L
                rows16 = idx_v[pl.ds(tok0, L)] * d_model
                msk16 = mask_v[pl.ds(tok0, L)]

                def batch_loads(j0):
                    cols = []
                    for j in range(j0, j0 + TB):
                        row_s = lane_bcast(rows16, j)
                        for s in range(n_slices):
                            cols.append(
                                plsc.load_gather(table_v, [row_s + soff[s]]))
                    return cols

                def batch_stores(j0, cols):
                    k = 0
                    for j in range(j0, j0 + TB):
                        msk_s = lane_bcast(msk16, j)
                        tbase = (tok0 + j) * d_model
                        for s in range(n_slices):
                            buf[pl.ds(tbase + s * L, L)] = cols[k] * msk_s
                            k += 1

                cols = batch_loads(0)
                for j0 in range(TB, L, TB):
                    nxt = batch_loads(j0)
                    batch_stores(j0 - TB, cols)
                    cols = nxt
                batch_stores(L - TB, cols)
            pltpu.async_copy(
                buf, out_hbm.at[pl.ds(start * d_model, chunk * d_model)], sem)

        for b in range(NBUF):
            start_loads(b, b)

        def outer(i, carry):
            for b in range(NBUF):
                ci = i * NBUF + b
                wait_loads(ci, b)

                @pl.when(ci >= NBUF)
                def _():
                    prev = base + (ci - NBUF) * chunk
                    pltpu.make_async_copy(
                        out_bufs[b],
                        out_hbm.at[pl.ds(prev * d_model, chunk * d_model)],
                        out_sems[b]).wait()

                do_chunk(ci, b, out_bufs[b], out_sems[b])

                @pl.when(ci + NBUF < n_chunks)
                def _():
                    start_loads(ci + NBUF, b)
            return carry

        lax.fori_loop(0, n_chunks // NBUF, outer, 0)
        for b in range(NBUF):
            last = base + (n_chunks - NBUF + b) * chunk
            pltpu.make_async_copy(
                out_bufs[b],
                out_hbm.at[pl.ds(last * d_model, chunk * d_model)],
                out_sems[b]).wait()

    return emb_kernel


def kernel(x, mask, emb_table):
    b, h = x.shape
    vocab, d_model = emb_table.shape
    n = b * h
    x_flat = x.T.reshape(n).astype(jnp.int32)
    mask_flat = mask.T.reshape(n).astype(jnp.float32)
    table_flat = emb_table.reshape(vocab * d_model).astype(jnp.float32)
    info = plsc.get_sparse_core_info()
    call = _build_sc_call(n, vocab, d_model, info.num_cores, info.num_subcores)
    out = call(x_flat, mask_flat, table_flat)
    return out.reshape(h, b, d_model).transpose(1, 0, 2)
